# Initial kernel scaffold; baseline (speedup 1.0000x reference)
import jax
import jax.numpy as jnp
from jax import lax
from jax.experimental import pallas as pl
from jax.experimental.pallas import tpu as pltpu
from jax.experimental.pallas import tpu_sc as plsc

B, T = 1024, 200
VOCAB, D = 100000, 64
N = B * T
EPS = 1e-5

NC, NS, L = 2, 16, 16
NW = NC * NS
ROWS_W = N // NW
CHUNK = 800
NCHUNK = ROWS_W // CHUNK
G = 100
NG = CHUNK // G
BROW_W = B // NW
STAT_ROWS = B // NS
JCOL = (T + L - 1) // L


def _iota16():
    return lax.iota(jnp.int32, L)


def _splat_i32(x):
    return jnp.full((L,), x, dtype=jnp.int32)


def _fast_rsqrt(x):
    i = plsc.bitcast(x, jnp.int32)
    i = jnp.int32(0x5F3759DF) - lax.shift_right_logical(i, 1)
    y = plsc.bitcast(i, jnp.float32)
    for _ in range(3):
        y = y * (1.5 - 0.5 * x * y * y)
    return y


def _sc_body(cat1_h, cat2_h, num_h, len_h, gb_h, emb1_h, emb2_h, out_h,
             idx1_v, idx2_v, rows1_v, rows2_v, nbuf_v, numc_v, len_v,
             gb_v, stat_v, pub_v, tmp_v, shared_v, sem):
    cid = lax.axis_index("c")
    sid = lax.axis_index("s")
    wid = sid * NC + cid
    iota = _iota16()

    pltpu.sync_copy(len_h, len_v)
    pltpu.sync_copy(gb_h, gb_v)

    def cnt_body(i, acc):
        lv = len_v[pl.ds(i * L, L)]
        lc = jnp.clip(lv, 0, T).astype(jnp.float32)
        return acc + lc
    cnt_acc = lax.fori_loop(0, B // L, cnt_body, jnp.zeros((L,), jnp.float32))
    cnt_s = jnp.sum(cnt_acc)

    r0 = sid * STAT_ROWS
    acc_s = jnp.zeros((L,), jnp.float32)
    acc_q = jnp.zeros((L,), jnp.float32)
    SUB = 16
    for sub in range(STAT_ROWS // SUB):
        pltpu.sync_copy(num_h.at[pl.ds((r0 + sub * SUB) * T, SUB * T)],
                        stat_v.at[pl.ds(0, SUB * T)])

        def row_body(r, carry):
            a_s, a_q = carry
            len_sp = plsc.load_gather(len_v, [_splat_i32(r0 + sub * SUB + r)])
            for j in range(JCOL):
                x = stat_v[pl.ds(r * T + j * L, L)]
                col = iota + (j * L)
                m = (col < len_sp) & (col < T)
                a_s = a_s + jnp.where(m, x, 0.0)
                a_q = a_q + jnp.where(m, x * x, 0.0)
            return a_s, a_q
        acc_s, acc_q = lax.fori_loop(0, SUB, row_body, (acc_s, acc_q))

    s_part = jnp.sum(acc_s)
    q_part = jnp.sum(acc_q)
    pvec = jnp.where(iota == 0, s_part, jnp.where(iota == 1, q_part, 0.0))
    pub_v[...] = pvec
    pltpu.sync_copy(pub_v, shared_v.at[pl.ds(sid * L, L)])
    plsc.subcore_barrier()
    pltpu.sync_copy(shared_v, stat_v.at[pl.ds(0, NS * L)])
    tot = jnp.zeros((L,), jnp.float32)
    for i in range(NS):
        tot = tot + stat_v[pl.ds(i * L, L)]
    tmp_v[...] = tot
    sum_sp = plsc.load_gather(tmp_v, [_splat_i32(0)])
    q_sp = plsc.load_gather(tmp_v, [_splat_i32(1)])
    cnt_sp = jnp.maximum(jnp.full((L,), cnt_s), 1.0)
    mean = sum_sp / cnt_sp
    var = q_sp / cnt_sp - mean * mean
    inv = _fast_rsqrt(var + EPS)
    gamma_sp = plsc.load_gather(gb_v, [_splat_i32(0)])
    beta_sp = plsc.load_gather(gb_v, [_splat_i32(1)])
    scale = gamma_sp * inv
    shift = beta_sp - mean * scale

    def chunk_body(t, _):
        rowbase = wid * ROWS_W + t * CHUNK
        pltpu.sync_copy(cat1_h.at[pl.ds(wid * (ROWS_W // G) + t * NG, NG)],
                        idx1_v)
        pltpu.sync_copy(cat2_h.at[pl.ds(wid * (ROWS_W // G) + t * NG, NG)],
                        idx2_v)
        copies = []
        for k in range(NG):
            copies.append(pltpu.async_copy(
                emb1_h.at[idx1_v.at[k]], rows1_v.at[pl.ds(k * G, G)], sem))
        for k in range(NG):
            copies.append(pltpu.async_copy(
                emb2_h.at[idx2_v.at[k]], rows2_v.at[pl.ds(k * G, G)], sem))

        pltpu.sync_copy(num_h.at[pl.ds(rowbase, CHUNK)],
                        numc_v.at[pl.ds(0, CHUNK)])
        for b in range(CHUNK // T):
            brow = wid * BROW_W + t * (CHUNK // T) + b
            len_sp = plsc.load_gather(len_v, [_splat_i32(brow)])
            for j in range(JCOL):
                x = numc_v[pl.ds(b * T + j * L, L)]
                col = iota + (j * L)
                m = col < len_sp
                val = jnp.where(m, x * scale + shift, x)
                ridx = iota + (b * T + j * L)
                if (j + 1) * L <= T:
                    plsc.store_scatter(nbuf_v, [ridx, _splat_i32(0)], val)
                else:
                    plsc.store_scatter(nbuf_v, [ridx, _splat_i32(0)], val,
                                       mask=col < T)

        for c in copies:
            c.wait()
        pltpu.sync_copy(rows1_v, out_h.at[pl.ds(rowbase, CHUNK), pl.ds(0, D)])
        pltpu.sync_copy(rows2_v, out_h.at[pl.ds(rowbase, CHUNK), pl.ds(D, D)])
        pltpu.sync_copy(nbuf_v,
                        out_h.at[pl.ds(rowbase, CHUNK), pl.ds(2 * D, 1)])
        return ()

    lax.fori_loop(0, NCHUNK, chunk_body, ())


@jax.jit
def _sc_feature_processor(cat1f, cat2f, num1f, lens, gb, emb1, emb2):
    mesh = plsc.VectorSubcoreMesh(core_axis_name="c", subcore_axis_name="s")
    kern = pl.kernel(
        _sc_body,
        out_type=jax.ShapeDtypeStruct((N, 2 * D + 1), jnp.float32),
        mesh=mesh,
        scratch_types=[
            pltpu.VMEM((NG, G), jnp.int32),
            pltpu.VMEM((NG, G), jnp.int32),
            pltpu.VMEM((CHUNK, D), jnp.float32),
            pltpu.VMEM((CHUNK, D), jnp.float32),
            pltpu.VMEM((CHUNK, 1), jnp.float32),
            pltpu.VMEM((CHUNK + L, ), jnp.float32),
            pltpu.VMEM((B,), jnp.int32),
            pltpu.VMEM((8,), jnp.float32),
            pltpu.VMEM((16 * T + L,), jnp.float32),
            pltpu.VMEM((L,), jnp.float32),
            pltpu.VMEM((L,), jnp.float32),
            pltpu.VMEM_SHARED((NS * L,), jnp.float32),
            pltpu.SemaphoreType.DMA,
        ],
    )
    return kern(cat1f, cat2f, num1f, lens, gb, emb1, emb2)


def kernel(event_time, seq_lens, cat1, cat2, num1, emb1, emb2, gamma, beta):
    cat1f = cat1.reshape(N // G, G).astype(jnp.int32)
    cat2f = cat2.reshape(N // G, G).astype(jnp.int32)
    num1f = num1.astype(jnp.float32).reshape(N)
    lens = seq_lens.astype(jnp.int32)
    gb = jnp.concatenate([gamma.astype(jnp.float32),
                          beta.astype(jnp.float32),
                          jnp.zeros((6,), jnp.float32)])
    out = _sc_feature_processor(cat1f, cat2f, num1f, lens, gb, emb1, emb2)
    return out.reshape(B, T, 2 * D + 1), event_time.astype(jnp.float32)


# speedup vs baseline: 2.5595x; 2.5595x over previous
import jax
import jax.numpy as jnp
from jax import lax
from jax.experimental import pallas as pl
from jax.experimental.pallas import tpu as pltpu
from jax.experimental.pallas import tpu_sc as plsc

B, T = 1024, 200
VOCAB, D = 100000, 64
N = B * T
EPS = 1e-5

NC, NS, L = 2, 16, 16
NW = NC * NS
ROWS_W = N // NW
CHUNK = 800
NCHUNK = ROWS_W // CHUNK
G = 100
NG = CHUNK // G
BROW_W = B // NW
STAT_ROWS = B // NS
JCOL = (T + L - 1) // L


def _iota16():
    return lax.iota(jnp.int32, L)


def _splat_i32(x):
    return jnp.full((L,), x, dtype=jnp.int32)


def _fast_rsqrt(x):
    i = plsc.bitcast(x, jnp.int32)
    i = jnp.int32(0x5F3759DF) - lax.shift_right_logical(i, 1)
    y = plsc.bitcast(i, jnp.float32)
    for _ in range(3):
        y = y * (1.5 - 0.5 * x * y * y)
    return y


def _sc_body(cat1_h, cat2_h, num_h, len_h, gb_h, emb1_h, emb2_h, out_h,
             idx1_v, idx2_v, rows1_v, rows2_v, nbuf_v, numc_v, len_v,
             gb_v, stat_v, pub_v, shared_v, sem):
    cid = lax.axis_index("c")
    sid = lax.axis_index("s")
    wid = sid * NC + cid
    iota = _iota16()

    pltpu.sync_copy(len_h, len_v)
    pltpu.sync_copy(gb_h, gb_v)

    def cnt_body(i, acc):
        lv = len_v[pl.ds(i * L, L)]
        lc = jnp.clip(lv, 0, T).astype(jnp.float32)
        return acc + lc
    cnt_acc = lax.fori_loop(0, B // L, cnt_body, jnp.zeros((L,), jnp.float32))
    cnt_s = jnp.sum(cnt_acc)

    r0 = sid * STAT_ROWS
    acc_s = jnp.zeros((L,), jnp.float32)
    acc_q = jnp.zeros((L,), jnp.float32)
    SUB = 16
    for sub in range(STAT_ROWS // SUB):
        pltpu.sync_copy(num_h.at[pl.ds((r0 + sub * SUB) * T, SUB * T)],
                        stat_v.at[pl.ds(0, SUB * T)])

        def row_body(r, carry):
            a_s, a_q = carry
            len_sp = plsc.load_gather(len_v, [_splat_i32(r0 + sub * SUB + r)])
            for j in range(JCOL):
                x = stat_v[pl.ds(r * T + j * L, L)]
                col = iota + (j * L)
                m = (col < len_sp) & (col < T)
                a_s = a_s + jnp.where(m, x, 0.0)
                a_q = a_q + jnp.where(m, x * x, 0.0)
            return a_s, a_q
        acc_s, acc_q = lax.fori_loop(0, SUB, row_body, (acc_s, acc_q))

    s_part = jnp.sum(acc_s)
    q_part = jnp.sum(acc_q)
    pvec = jnp.where(iota == 0, s_part, jnp.where(iota == 1, q_part, 0.0))
    pub_v[...] = pvec
    pltpu.sync_copy(pub_v, shared_v.at[pl.ds(sid * L, L)])
    plsc.subcore_barrier()
    pltpu.sync_copy(shared_v, stat_v.at[pl.ds(0, NS * L)])
    tot = jnp.zeros((L,), jnp.float32)
    for i in range(NS):
        tot = tot + stat_v[pl.ds(i * L, L)]
    def _lane(v, k):
        return jnp.full((L,), jnp.sum(jnp.where(iota == k, v, 0.0)))

    sum_sp = _lane(tot, 0)
    q_sp = _lane(tot, 1)
    cnt_sp = jnp.maximum(jnp.full((L,), cnt_s), 1.0)
    mean = sum_sp / cnt_sp
    var = q_sp / cnt_sp - mean * mean
    inv = _fast_rsqrt(var + EPS)
    gbvec = gb_v[...]
    gamma_sp = _lane(gbvec, 0)
    beta_sp = _lane(gbvec, 1)
    scale = gamma_sp * inv
    shift = beta_sp - mean * scale

    def chunk_body(t, _):
        rowbase = wid * ROWS_W + t * CHUNK
        pltpu.sync_copy(cat1_h.at[pl.ds(wid * (ROWS_W // G) + t * NG, NG)],
                        idx1_v)
        pltpu.sync_copy(cat2_h.at[pl.ds(wid * (ROWS_W // G) + t * NG, NG)],
                        idx2_v)
        copies = []
        for k in range(NG):
            copies.append(pltpu.async_copy(
                emb1_h.at[idx1_v.at[k]], rows1_v.at[pl.ds(k * G, G)], sem))
        for k in range(NG):
            copies.append(pltpu.async_copy(
                emb2_h.at[idx2_v.at[k]], rows2_v.at[pl.ds(k * G, G)], sem))

        pltpu.sync_copy(num_h.at[pl.ds(rowbase, CHUNK)],
                        numc_v.at[pl.ds(0, CHUNK)])
        for b in range(CHUNK // T):
            brow = wid * BROW_W + t * (CHUNK // T) + b
            len_sp = plsc.load_gather(len_v, [_splat_i32(brow)])
            for j in range(JCOL):
                x = numc_v[pl.ds(b * T + j * L, L)]
                col = iota + (j * L)
                m = col < len_sp
                val = jnp.where(m, x * scale + shift, x)
                ridx = iota + (b * T + j * L)
                if (j + 1) * L <= T:
                    plsc.store_scatter(nbuf_v, [ridx, _splat_i32(0)], val)
                else:
                    plsc.store_scatter(nbuf_v, [ridx, _splat_i32(0)], val,
                                       mask=col < T)

        for c in copies:
            c.wait()
        pltpu.sync_copy(rows1_v, out_h.at[pl.ds(rowbase, CHUNK), pl.ds(0, D)])
        pltpu.sync_copy(rows2_v, out_h.at[pl.ds(rowbase, CHUNK), pl.ds(D, D)])
        pltpu.sync_copy(nbuf_v,
                        out_h.at[pl.ds(rowbase, CHUNK), pl.ds(2 * D, 1)])
        return ()

    lax.fori_loop(0, NCHUNK, chunk_body, ())


@jax.jit
def _sc_feature_processor(cat1f, cat2f, num1f, lens, gb, emb1, emb2):
    mesh = plsc.VectorSubcoreMesh(core_axis_name="c", subcore_axis_name="s")
    kern = pl.kernel(
        _sc_body,
        out_type=jax.ShapeDtypeStruct((N, 2 * D + 1), jnp.float32),
        mesh=mesh,
        compiler_params=pltpu.CompilerParams(use_tc_tiling_on_sc=False,
                                             needs_layout_passes=False),
        scratch_types=[
            pltpu.VMEM((NG, G), jnp.int32),
            pltpu.VMEM((NG, G), jnp.int32),
            pltpu.VMEM((CHUNK, D), jnp.float32),
            pltpu.VMEM((CHUNK, D), jnp.float32),
            pltpu.VMEM((CHUNK, 1), jnp.float32),
            pltpu.VMEM((CHUNK + L, ), jnp.float32),
            pltpu.VMEM((B,), jnp.int32),
            pltpu.VMEM((L,), jnp.float32),
            pltpu.VMEM((16 * T + L,), jnp.float32),
            pltpu.VMEM((L,), jnp.float32),
            pltpu.VMEM_SHARED((NS * L,), jnp.float32),
            pltpu.SemaphoreType.DMA,
        ],
    )
    return kern(cat1f, cat2f, num1f, lens, gb, emb1, emb2)


def kernel(event_time, seq_lens, cat1, cat2, num1, emb1, emb2, gamma, beta):
    cat1f = cat1.reshape(N // G, G).astype(jnp.int32)
    cat2f = cat2.reshape(N // G, G).astype(jnp.int32)
    num1f = num1.astype(jnp.float32).reshape(N)
    lens = seq_lens.astype(jnp.int32)
    gb = jnp.concatenate([gamma.astype(jnp.float32),
                          beta.astype(jnp.float32),
                          jnp.zeros((14,), jnp.float32)])
    out = _sc_feature_processor(cat1f, cat2f, num1f, lens, gb, emb1, emb2)
    return out.reshape(B, T, 2 * D + 1), event_time.astype(jnp.float32)
